# baseline (speedup 1.0000x reference)
import functools

import jax
import jax.numpy as jnp
from jax import lax
from jax.experimental import pallas as pl
from jax.experimental.pallas import tpu as pltpu
from jax.experimental.pallas import tpu_sc as plsc

N_DIM = 256
E_DIM = 128
A_DIM = 64
AXIS = 4
NLOC = 10000
NALL = 12000
NNEI = 16
NEDGE = 160000
NANGLE = 160000
DYN_E = NNEI / 10.0
DYN_A = 16 / 10.0

EBLK = 1280
NBLK = 1000


def _silu(x):
    return x * (1.0 / (1.0 + jnp.exp(-x)))


def _full_spec(shape):
    return pl.BlockSpec(shape, lambda i: tuple(0 for _ in shape))


def _row_spec(blk, shape):
    return pl.BlockSpec((blk,) + shape[1:], lambda i: (i,) + tuple(0 for _ in shape[1:]))


def _edge_body(ni, nn, ee, sw, h2, A1, A2, A3, bne, B1, B2, B3, bes, eres,
               neu, epart, w3):
    x_ni = ni[...]
    x_nn = nn[...]
    x_ee = ee[...]
    s = sw[...]
    dot = functools.partial(jnp.dot, preferred_element_type=jnp.float32)
    pre_u = dot(x_ni, A1[...]) + dot(x_nn, A2[...]) + dot(x_ee, A3[...]) + bne[...]
    neu[...] = _silu(pre_u) * s
    pre_e = dot(x_ni, B1[...]) + dot(x_nn, B2[...]) + dot(x_ee, B3[...]) + bes[...]
    epart[...] = x_ee + eres[...] * _silu(pre_e)
    w3[...] = h2[...] * s


def _tc_edge_mlp(node_i, nei, edge_ebd, sw2, h2, W_ne, b_ne, W_es, b_es, e_res0):
    grid = (NEDGE // EBLK,)
    out_shapes = (
        jax.ShapeDtypeStruct((NEDGE, N_DIM), jnp.float32),
        jax.ShapeDtypeStruct((NEDGE, E_DIM), jnp.float32),
        jax.ShapeDtypeStruct((NEDGE, 3), jnp.float32),
    )
    A1, A2, A3 = W_ne[:N_DIM], W_ne[N_DIM:2 * N_DIM], W_ne[2 * N_DIM:]
    B1, B2, B3 = W_es[:N_DIM], W_es[N_DIM:2 * N_DIM], W_es[2 * N_DIM:]
    return pl.pallas_call(
        _edge_body,
        grid=grid,
        in_specs=[
            _row_spec(EBLK, (NEDGE, N_DIM)),
            _row_spec(EBLK, (NEDGE, N_DIM)),
            _row_spec(EBLK, (NEDGE, E_DIM)),
            _row_spec(EBLK, (NEDGE, 1)),
            _row_spec(EBLK, (NEDGE, 3)),
            _full_spec((N_DIM, N_DIM)),
            _full_spec((N_DIM, N_DIM)),
            _full_spec((E_DIM, N_DIM)),
            _full_spec((1, N_DIM)),
            _full_spec((N_DIM, E_DIM)),
            _full_spec((N_DIM, E_DIM)),
            _full_spec((E_DIM, E_DIM)),
            _full_spec((1, E_DIM)),
            _full_spec((1, E_DIM)),
        ],
        out_specs=[
            _row_spec(EBLK, (NEDGE, N_DIM)),
            _row_spec(EBLK, (NEDGE, E_DIM)),
            _row_spec(EBLK, (NEDGE, 3)),
        ],
        out_shape=out_shapes,
    )(node_i, nei, edge_ebd, sw2, h2, A1, A2, A3, b_ne[None, :],
      B1, B2, B3, b_es[None, :], e_res0[None, :])


def _angle_body(ab, na, ik, ij, asw, C1, C2, C3, C4, bea, D1, D2, D3, D4, bas,
                ares, eaw, aupd):
    x_ab = ab[...]
    x_na = na[...]
    x_ik = ik[...]
    x_ij = ij[...]
    dot = functools.partial(jnp.dot, preferred_element_type=jnp.float32)
    pre_e = (dot(x_ab, C1[...]) + dot(x_na, C2[...]) + dot(x_ik, C3[...])
             + dot(x_ij, C4[...]) + bea[...])
    eaw[...] = _silu(pre_e) * asw[...]
    pre_a = (dot(x_ab, D1[...]) + dot(x_na, D2[...]) + dot(x_ik, D3[...])
             + dot(x_ij, D4[...]) + bas[...])
    aupd[...] = x_ab + ares[...] * _silu(pre_a)


def _tc_angle_mlp(angle_ebd, node_a, edge_ik, edge_ij, a_sw2, W_ea1, b_ea1,
                  W_as, b_as, a_res0):
    grid = (NANGLE // EBLK,)
    C1 = W_ea1[:A_DIM]
    C2 = W_ea1[A_DIM:A_DIM + N_DIM]
    C3 = W_ea1[A_DIM + N_DIM:A_DIM + N_DIM + E_DIM]
    C4 = W_ea1[A_DIM + N_DIM + E_DIM:]
    D1 = W_as[:A_DIM]
    D2 = W_as[A_DIM:A_DIM + N_DIM]
    D3 = W_as[A_DIM + N_DIM:A_DIM + N_DIM + E_DIM]
    D4 = W_as[A_DIM + N_DIM + E_DIM:]
    return pl.pallas_call(
        _angle_body,
        grid=grid,
        in_specs=[
            _row_spec(EBLK, (NANGLE, A_DIM)),
            _row_spec(EBLK, (NANGLE, N_DIM)),
            _row_spec(EBLK, (NANGLE, E_DIM)),
            _row_spec(EBLK, (NANGLE, E_DIM)),
            _row_spec(EBLK, (NANGLE, 1)),
            _full_spec((A_DIM, E_DIM)),
            _full_spec((N_DIM, E_DIM)),
            _full_spec((E_DIM, E_DIM)),
            _full_spec((E_DIM, E_DIM)),
            _full_spec((1, E_DIM)),
            _full_spec((A_DIM, A_DIM)),
            _full_spec((N_DIM, A_DIM)),
            _full_spec((E_DIM, A_DIM)),
            _full_spec((E_DIM, A_DIM)),
            _full_spec((1, A_DIM)),
            _full_spec((1, A_DIM)),
        ],
        out_specs=[
            _row_spec(EBLK, (NANGLE, E_DIM)),
            _row_spec(EBLK, (NANGLE, A_DIM)),
        ],
        out_shape=(
            jax.ShapeDtypeStruct((NANGLE, E_DIM), jnp.float32),
            jax.ShapeDtypeStruct((NANGLE, A_DIM), jnp.float32),
        ),
    )(angle_ebd, node_a, edge_ik, edge_ij, a_sw2, C1, C2, C3, C4, b_ea1[None, :],
      D1, D2, D3, D4, b_as[None, :], a_res0[None, :])


def _node_body(xr, her, hnr, msgr, Wns, bns, Wsym, bsym, nr0, nr1, nr2, out):
    x = xr[...]
    dot = functools.partial(jnp.dot, preferred_element_type=jnp.float32)
    node_self = _silu(dot(x, Wns[...]) + bns[...])
    F = 1.0 / (DYN_E * 3.0)
    pre = jnp.zeros_like(x) + bsym[...]
    he = [her[c] for c in range(3)]
    hn = [hnr[c] for c in range(3)]
    for a in range(AXIS):
        ge_a = (he[0][:, a:a + 1] * he[0] + he[1][:, a:a + 1] * he[1]
                + he[2][:, a:a + 1] * he[2]) * F
        pre += dot(ge_a, Wsym[a * E_DIM:(a + 1) * E_DIM, :])
        gn_a = (hn[0][:, a:a + 1] * hn[0] + hn[1][:, a:a + 1] * hn[1]
                + hn[2][:, a:a + 1] * hn[2]) * F
        base = AXIS * E_DIM + a * N_DIM
        pre += dot(gn_a, Wsym[base:base + N_DIM, :])
    node_sym = _silu(pre)
    out[...] = (x + nr0[...] * node_self + nr1[...] * node_sym
                + nr2[...] * (msgr[...] * (1.0 / DYN_E)))


def _tc_node(node_ebd, h2g2_e, h2g2_n, msg, W_ns, b_ns, W_sym, b_sym,
             n_res0, n_res1, n_res2):
    grid = (NLOC // NBLK,)
    return pl.pallas_call(
        _node_body,
        grid=grid,
        in_specs=[
            _row_spec(NBLK, (NLOC, N_DIM)),
            pl.BlockSpec((3, NBLK, E_DIM), lambda i: (0, i, 0)),
            pl.BlockSpec((3, NBLK, N_DIM), lambda i: (0, i, 0)),
            _row_spec(NBLK, (NLOC, N_DIM)),
            _full_spec((N_DIM, N_DIM)),
            _full_spec((1, N_DIM)),
            _full_spec(((N_DIM + E_DIM) * AXIS, N_DIM)),
            _full_spec((1, N_DIM)),
            _full_spec((1, N_DIM)),
            _full_spec((1, N_DIM)),
            _full_spec((1, N_DIM)),
        ],
        out_specs=[_row_spec(NBLK, (NLOC, N_DIM))],
        out_shape=(jax.ShapeDtypeStruct((NLOC, N_DIM), jnp.float32),),
    )(node_ebd, h2g2_e, h2g2_n, msg, W_ns, b_ns[None, :], W_sym, b_sym[None, :],
      n_res0[None, :], n_res1[None, :], n_res2[None, :])[0]


def _fin_body(ep, red, W, b, eres, out):
    dot = functools.partial(jnp.dot, preferred_element_type=jnp.float32)
    pre = dot(red[...] * (DYN_A ** -0.5), W[...]) + b[...]
    out[...] = ep[...] + eres[...] * _silu(pre)


def _tc_edge_fin(e_part, reduced, W_ea2, b_ea2, e_res1):
    grid = (NEDGE // EBLK,)
    return pl.pallas_call(
        _fin_body,
        grid=grid,
        in_specs=[
            _row_spec(EBLK, (NEDGE, E_DIM)),
            _row_spec(EBLK, (NEDGE, E_DIM)),
            _full_spec((E_DIM, E_DIM)),
            _full_spec((1, E_DIM)),
            _full_spec((1, E_DIM)),
        ],
        out_specs=[_row_spec(EBLK, (NEDGE, E_DIM))],
        out_shape=(jax.ShapeDtypeStruct((NEDGE, E_DIM), jnp.float32),),
    )(e_part, reduced, W_ea2, b_ea2[None, :], e_res1[None, :])[0]


_C = 128
_NP = 163840
_NW = 32
_CHN = 3 * (_NP // _C)
_CHE = 2 * (_NP // _C)
_CHN_W = _CHN // _NW
_CHE_W = _CHE // _NW


def _sc_gather(node_table, edge_table, idx_n, idx_e):
    mesh = plsc.VectorSubcoreMesh(core_axis_name="c", subcore_axis_name="s")

    @functools.partial(
        pl.kernel,
        out_type=(jax.ShapeDtypeStruct((3 * _NP, N_DIM), jnp.float32),
                  jax.ShapeDtypeStruct((2 * _NP, E_DIM), jnp.float32)),
        mesh=mesh,
        scratch_types=[
            pltpu.VMEM((_CHN_W, _C), jnp.int32),
            pltpu.VMEM((_CHE_W, _C), jnp.int32),
            pltpu.VMEM((_C, N_DIM), jnp.float32),
            pltpu.VMEM((_C, N_DIM), jnp.float32),
            pltpu.VMEM((_C, E_DIM), jnp.float32),
            pltpu.VMEM((_C, E_DIM), jnp.float32),
            pltpu.SemaphoreType.DMA,
            pltpu.SemaphoreType.DMA,
        ],
    )
    def k(ntab, etab, idxn, idxe, outn, oute,
          idxn_v, idxe_v, nb0, nb1, eb0, eb1, s0, s1):
        cid = lax.axis_index("c")
        sid = lax.axis_index("s")
        wid = sid * 2 + cid
        pltpu.sync_copy(idxn.at[pl.ds(wid * _CHN_W, _CHN_W)], idxn_v)
        pltpu.sync_copy(idxe.at[pl.ds(wid * _CHE_W, _CHE_W)], idxe_v)

        def run_set(tab, idx_v, out, bufs, sems, n_chunks, base):
            pltpu.async_copy(tab.at[idx_v.at[0]], bufs[0], sems[0])
            pltpu.async_copy(tab.at[idx_v.at[1]], bufs[1], sems[1])

            @pl.loop(0, n_chunks, step=2)
            def _(j):
                for b in range(2):
                    jj = j + b
                    pltpu.make_async_copy(tab.at[idx_v.at[0]], bufs[b],
                                          sems[b]).wait()
                    pltpu.sync_copy(bufs[b],
                                    out.at[pl.ds((base + jj) * _C, _C)])

                    @pl.when(jj + 2 < n_chunks)
                    def _():
                        pltpu.async_copy(tab.at[idx_v.at[jj + 2]], bufs[b],
                                         sems[b])

        run_set(ntab, idxn_v, outn, (nb0, nb1), (s0, s1), _CHN_W,
                wid * _CHN_W)
        run_set(etab, idxe_v, oute, (eb0, eb1), (s0, s1), _CHE_W,
                wid * _CHE_W)

    return k(node_table, edge_table, idx_n, idx_e)


def _gather_rows(table, idx):
    return jnp.take(table, idx, axis=0)


def _segsum(data, owner, num):
    return jax.ops.segment_sum(data, owner, num_segments=num)


def kernel(node_ebd_ext, edge_ebd, h2, angle_ebd, nlist, nlist_mask, sw,
           a_nlist, a_nlist_mask, a_sw, edge_index, angle_index, W_ns, b_ns,
           W_sym, b_sym, W_ne, b_ne, W_es, b_es, W_ea1, b_ea1, W_ea2, b_ea2,
           W_as, b_as, n_res0, n_res1, n_res2, e_res0, e_res1, a_res0):
    node_ext_flat = node_ebd_ext.reshape(-1, N_DIM)
    node_ebd = node_ext_flat[:NLOC]
    n2e = edge_index[0]
    nx2e = edge_index[1]
    n2a = angle_index[0]
    eij2a = angle_index[1]
    eik2a = angle_index[2]

    pad = _NP - NEDGE

    def _pad_idx(ix):
        return jnp.pad(ix, (0, pad))

    idx_n = jnp.concatenate(
        [_pad_idx(nx2e), _pad_idx(n2e), _pad_idx(n2a)]).reshape(_CHN, _C)
    idx_e = jnp.concatenate(
        [_pad_idx(eik2a), _pad_idx(eij2a)]).reshape(_CHE, _C)
    out_n, out_e = _sc_gather(node_ext_flat, edge_ebd, idx_n, idx_e)
    nei = out_n[:NEDGE]
    node_i = out_n[_NP:_NP + NEDGE]
    node_a = out_n[2 * _NP:2 * _NP + NANGLE]
    edge_ik = out_e[:NANGLE]
    edge_ij = out_e[_NP:_NP + NANGLE]

    neu, e_part, w3 = _tc_edge_mlp(node_i, nei, edge_ebd, sw[:, None], h2,
                                   W_ne, b_ne, W_es, b_es, e_res0)
    eaw, a_updated = _tc_angle_mlp(angle_ebd, node_a, edge_ik, edge_ij,
                                   a_sw[:, None], W_ea1, b_ea1, W_as, b_as,
                                   a_res0)

    h2g2_e = jnp.stack([_segsum(w3[:, c:c + 1] * edge_ebd, n2e, NLOC)
                        for c in range(3)], axis=0)
    h2g2_n = jnp.stack([_segsum(w3[:, c:c + 1] * nei, n2e, NLOC)
                        for c in range(3)], axis=0)
    msg = _segsum(neu, n2e, NLOC)
    reduced = _segsum(eaw, eij2a, NEDGE)

    n_updated = _tc_node(node_ebd, h2g2_e, h2g2_n, msg, W_ns, b_ns, W_sym,
                         b_sym, n_res0, n_res1, n_res2)
    e_updated = _tc_edge_fin(e_part, reduced, W_ea2, b_ea2, e_res1)

    return (n_updated.reshape(1, NLOC, N_DIM), e_updated, a_updated)


# speedup vs baseline: 1.6358x; 1.6358x over previous
import functools

import jax
import jax.numpy as jnp
from jax import lax
from jax.experimental import pallas as pl
from jax.experimental.pallas import tpu as pltpu
from jax.experimental.pallas import tpu_sc as plsc

N_DIM = 256
E_DIM = 128
A_DIM = 64
AXIS = 4
NLOC = 10000
NALL = 12000
NNEI = 16
NEDGE = 160000
NANGLE = 160000
DYN_E = NNEI / 10.0
DYN_A = 16 / 10.0

EBLK = 1280
NBLK = 1000

_C = 128
_NP = 163840
_NW = 32
_NCH = _NP // _C
_CHN = 3 * _NCH
_CHE = 2 * _NCH
_CHN_W = _CHN // _NW
_CHE_W = _CHE // _NW
_CH_T = _NCH // 16

_RN = 10240
_TZ = _RN // 16
_DUMP = 10200
_SEG4 = 10000
_PK = N_DIM // 2

_BF = jnp.bfloat16
_F32 = jnp.float32


def _silu(x):
    return x * (1.0 / (1.0 + jnp.exp(-x)))


def _unpack2(x):
    lo = lax.bitcast_convert_type(x << 16, _F32)
    hi = lax.bitcast_convert_type(x & jnp.int32(-65536), _F32)
    return lo, hi


def _full_spec(shape):
    return pl.BlockSpec(shape, lambda i: tuple(0 for _ in shape))


def _row_spec(blk, shape, off=0):
    return pl.BlockSpec((blk,) + shape[1:],
                        lambda i: (i + off,) + tuple(0 for _ in shape[1:]))


def _sc_gather(node_pack, edge_table, idx_n, idx_e):
    mesh = plsc.VectorSubcoreMesh(core_axis_name="c", subcore_axis_name="s")

    @functools.partial(
        pl.kernel,
        out_type=(jax.ShapeDtypeStruct((3 * _NP, _PK), jnp.int32),
                  jax.ShapeDtypeStruct((2 * _NP, E_DIM), _F32)),
        mesh=mesh,
        scratch_types=(
            [pltpu.VMEM((_CHN_W, _C), jnp.int32),
             pltpu.VMEM((_CHE_W, _C), jnp.int32)]
            + [pltpu.VMEM((_C, _PK), jnp.int32) for _ in range(3)]
            + [pltpu.VMEM((_C, E_DIM), _F32) for _ in range(3)]
            + [pltpu.SemaphoreType.DMA for _ in range(6)]
        ),
    )
    def k(ntab, etab, idxn, idxe, outn, oute,
          idxn_v, idxe_v, nb0, nb1, nb2, eb0, eb1, eb2,
          g0, g1, g2, s0, s1, s2):
        cid = lax.axis_index("c")
        sid = lax.axis_index("s")
        wid = sid * 2 + cid
        gsems = (g0, g1, g2)
        ssems = (s0, s1, s2)
        pltpu.sync_copy(idxn.at[pl.ds(wid * _CHN_W, _CHN_W)], idxn_v)
        pltpu.sync_copy(idxe.at[pl.ds(wid * _CHE_W, _CHE_W)], idxe_v)

        def run_set(tab, idx_v, out, bufs, n_chunks, base):
            @pl.loop(0, n_chunks + 3, step=3)
            def _(j0):
                for b in range(3):
                    jj = j0 + b

                    @pl.when((jj >= 3) & (jj < n_chunks))
                    def _():
                        pltpu.make_async_copy(
                            bufs[b], out.at[pl.ds(base * _C, _C)],
                            ssems[b]).wait()

                    @pl.when(jj < n_chunks)
                    def _():
                        pltpu.async_copy(tab.at[idx_v.at[jj]], bufs[b],
                                         gsems[b])

                    kk = jj - 2
                    b2 = (b + 1) % 3

                    @pl.when((kk >= 0) & (kk < n_chunks))
                    def _():
                        pltpu.make_async_copy(tab.at[pl.ds(0, _C)], bufs[b2],
                                              gsems[b2]).wait()
                        pltpu.async_copy(
                            bufs[b2], out.at[pl.ds((base + kk) * _C, _C)],
                            ssems[b2])

            for b in range(3):
                pltpu.make_async_copy(
                    bufs[b], out.at[pl.ds(base * _C, _C)],
                    ssems[b]).wait()

        run_set(ntab, idxn_v, outn, (nb0, nb1, nb2), _CHN_W, wid * _CHN_W)
        run_set(etab, idxe_v, oute, (eb0, eb1, eb2), _CHE_W, wid * _CHE_W)

    return k(node_pack, edge_table, idx_n, idx_e)


def _sc_segsum_node(neu, fh0, fh1, fh2, idx2d, zer):
    mesh = plsc.VectorSubcoreMesh(core_axis_name="c", subcore_axis_name="s")

    @functools.partial(
        pl.kernel,
        out_type=(jax.ShapeDtypeStruct((_RN, E_DIM), _F32),
                  jax.ShapeDtypeStruct((_RN, E_DIM), _F32),
                  jax.ShapeDtypeStruct((3, _RN, E_DIM), _F32),
                  jax.ShapeDtypeStruct((3, _RN, E_DIM), _F32),
                  jax.ShapeDtypeStruct((3, _RN, E_DIM), _F32)),
        mesh=mesh,
        scratch_types=(
            [pltpu.VMEM_SHARED((_RN, E_DIM), _F32),
             pltpu.VMEM((_CH_T, _C), jnp.int32),
             pltpu.VMEM((_C, E_DIM), _F32),
             pltpu.VMEM((_C, E_DIM), _F32),
             pltpu.SemaphoreType.DMA,
             pltpu.SemaphoreType.DMA]
        ),
    )
    def k(neu_r, f0_r, f1_r, f2_r, idx_r, zer_r,
          msga, msgb, hge, hgna, hgnb,
          accum, idx_v, db0, db1, m0, m1):
        cid = lax.axis_index("c")
        sid = lax.axis_index("s")
        bufs = (db0, db1)
        sems = (m0, m1)
        pltpu.sync_copy(idx_r.at[pl.ds(sid * _CH_T, _CH_T)], idx_v)
        sl = pl.ds(sid * _TZ, _TZ)

        def fire(data, grp, jj, b):
            pltpu.async_copy(
                data.at[pl.ds((sid * _CH_T + jj) * _C, _C),
                        pl.ds(grp * E_DIM, E_DIM)],
                bufs[b], sems[b])

        def scat_loop(data, grp):
            fire(data, grp, 0, 0)
            fire(data, grp, 1, 1)

            @pl.loop(0, _CH_T, step=2)
            def _(j):
                for b in range(2):
                    jj = j + b
                    pltpu.make_async_copy(
                        data.at[pl.ds(0, _C), pl.ds(grp * E_DIM, E_DIM)],
                        bufs[b], sems[b]).wait()
                    pltpu.sync_copy(bufs[b], accum.at[idx_v.at[jj]],
                                    add=True)

                    @pl.when(jj + 2 < _CH_T)
                    def _():
                        fire(data, grp, jj + 2, b)

        jobs0 = [(neu_r, 0, msga.at[sl]),
                 (f0_r, 0, hge.at[0, sl]),
                 (f1_r, 0, hge.at[1, sl]),
                 (f2_r, 0, hge.at[2, sl]),
                 (f0_r, 1, hgna.at[0, sl]),
                 (f0_r, 2, hgnb.at[0, sl])]
        jobs1 = [(neu_r, 1, msgb.at[sl]),
                 (f1_r, 1, hgna.at[1, sl]),
                 (f2_r, 1, hgna.at[2, sl]),
                 (f1_r, 2, hgnb.at[1, sl]),
                 (f2_r, 2, hgnb.at[2, sl])]
        for step in range(6):
            pltpu.sync_copy(zer_r.at[sl], accum.at[sl])
            plsc.subcore_barrier()

            @pl.when(cid == 0)
            def _(step=step):
                scat_loop(jobs0[step][0], jobs0[step][1])

            if step < 5:
                @pl.when(cid == 1)
                def _(step=step):
                    scat_loop(jobs1[step][0], jobs1[step][1])

            plsc.subcore_barrier()

            @pl.when(cid == 0)
            def _(step=step):
                pltpu.sync_copy(accum.at[sl], jobs0[step][2])

            if step < 5:
                @pl.when(cid == 1)
                def _(step=step):
                    pltpu.sync_copy(accum.at[sl], jobs1[step][2])

            plsc.subcore_barrier()

    return k(neu, fh0, fh1, fh2, idx2d, zer)


def _sc_segsum_edge(eaw, idx16, zer):
    mesh = plsc.VectorSubcoreMesh(core_axis_name="c", subcore_axis_name="s")

    @functools.partial(
        pl.kernel,
        out_type=jax.ShapeDtypeStruct((NEDGE, E_DIM), _F32),
        mesh=mesh,
        scratch_types=(
            [pltpu.VMEM_SHARED((_RN, E_DIM), _F32),
             pltpu.VMEM((_CH_T, _C), jnp.int32),
             pltpu.VMEM((_C, E_DIM), _F32),
             pltpu.VMEM((_C, E_DIM), _F32),
             pltpu.SemaphoreType.DMA,
             pltpu.SemaphoreType.DMA]
        ),
    )
    def k(eaw_r, idx_r, zer_r, out, accum, idx_v, db0, db1, m0, m1):
        cid = lax.axis_index("c")
        sid = lax.axis_index("s")
        bufs = (db0, db1)
        sems = (m0, m1)
        sl = pl.ds(sid * _TZ, _TZ)

        def fire(jj, b):
            pltpu.async_copy(
                eaw_r.at[pl.ds((sid * _CH_T + jj) * _C, _C)],
                bufs[b], sems[b])

        def run_pass(p):
            pltpu.sync_copy(zer_r.at[sl], accum.at[sl])
            pltpu.sync_copy(
                idx_r.at[pl.ds((p * 16 + sid) * _CH_T, _CH_T)], idx_v)
            plsc.subcore_barrier()
            fire(0, 0)
            fire(1, 1)

            @pl.loop(0, _CH_T, step=2)
            def _(j):
                for b in range(2):
                    jj = j + b
                    pltpu.make_async_copy(eaw_r.at[pl.ds(0, _C)],
                                          bufs[b], sems[b]).wait()
                    pltpu.sync_copy(bufs[b], accum.at[idx_v.at[jj]],
                                    add=True)

                    @pl.when(jj + 2 < _CH_T)
                    def _():
                        fire(jj + 2, b)

            plsc.subcore_barrier()

            @pl.when(sid < 5)
            def _():
                pltpu.sync_copy(
                    accum.at[pl.ds(sid * 2000, 2000)],
                    out.at[pl.ds(p * _SEG4 + sid * 2000, 2000)])

            plsc.subcore_barrier()

        @pl.when(cid == 0)
        def _():
            for p in range(0, 16, 2):
                run_pass(p)

        @pl.when(cid == 1)
        def _():
            for p in range(1, 16, 2):
                run_pass(p)

    return k(eaw, idx16, zer)


def _edge_body(ni, nn, ee, sw, h2,
               A1e, A1o, A2e, A2o, A3, bne,
               B1e, B1o, B2e, B2o, B3, bes, eres,
               neu, epart, f0, f1, f2):
    ni_e, ni_o = _unpack2(ni[...])
    nn_e, nn_o = _unpack2(nn[...])
    x_ee32 = ee[...]
    x_ee = x_ee32.astype(_BF)
    nib_e, nib_o = ni_e.astype(_BF), ni_o.astype(_BF)
    nnb_e, nnb_o = nn_e.astype(_BF), nn_o.astype(_BF)
    s = sw[...]
    dot = functools.partial(jnp.dot, preferred_element_type=_F32)
    pre_u = (dot(nib_e, A1e[...]) + dot(nib_o, A1o[...])
             + dot(nnb_e, A2e[...]) + dot(nnb_o, A2o[...])
             + dot(x_ee, A3[...]) + bne[...])
    neu[...] = _silu(pre_u) * s
    pre_e = (dot(nib_e, B1e[...]) + dot(nib_o, B1o[...])
             + dot(nnb_e, B2e[...]) + dot(nnb_o, B2o[...])
             + dot(x_ee, B3[...]) + bes[...])
    epart[...] = x_ee32 + eres[...] * _silu(pre_e)
    cat = jnp.concatenate([x_ee32, nn_e, nn_o], axis=1)
    h = h2[...]
    for c, fr in enumerate((f0, f1, f2)):
        fr[...] = (h[:, c:c + 1] * s) * cat


def _tc_edge_mlp(out_n, edge_ebd, sw2, h2, Wne_p, b_ne, Wes_p, b_es, e_res0):
    grid = (NEDGE // EBLK,)
    nblk = _NP // EBLK
    return pl.pallas_call(
        _edge_body,
        grid=grid,
        in_specs=[
            _row_spec(EBLK, (3 * _NP, _PK), off=nblk),
            _row_spec(EBLK, (3 * _NP, _PK)),
            _row_spec(EBLK, (NEDGE, E_DIM)),
            _row_spec(EBLK, (NEDGE, 1)),
            _row_spec(EBLK, (NEDGE, 3)),
            _full_spec((_PK, N_DIM)),
            _full_spec((_PK, N_DIM)),
            _full_spec((_PK, N_DIM)),
            _full_spec((_PK, N_DIM)),
            _full_spec((E_DIM, N_DIM)),
            _full_spec((1, N_DIM)),
            _full_spec((_PK, E_DIM)),
            _full_spec((_PK, E_DIM)),
            _full_spec((_PK, E_DIM)),
            _full_spec((_PK, E_DIM)),
            _full_spec((E_DIM, E_DIM)),
            _full_spec((1, E_DIM)),
            _full_spec((1, E_DIM)),
        ],
        out_specs=[
            _row_spec(EBLK, (_NP, N_DIM)),
            _row_spec(EBLK, (_NP, E_DIM)),
            _row_spec(EBLK, (_NP, 3 * E_DIM)),
            _row_spec(EBLK, (_NP, 3 * E_DIM)),
            _row_spec(EBLK, (_NP, 3 * E_DIM)),
        ],
        out_shape=(
            jax.ShapeDtypeStruct((_NP, N_DIM), _F32),
            jax.ShapeDtypeStruct((_NP, E_DIM), _F32),
            jax.ShapeDtypeStruct((_NP, 3 * E_DIM), _F32),
            jax.ShapeDtypeStruct((_NP, 3 * E_DIM), _F32),
            jax.ShapeDtypeStruct((_NP, 3 * E_DIM), _F32),
        ),
    )(out_n, out_n, edge_ebd, sw2, h2, *Wne_p, b_ne[None, :],
      *Wes_p, b_es[None, :], e_res0[None, :])


def _angle_body(ab, na, ik, ij, asw,
                C1, C2e, C2o, C3, C4, bea,
                D1, D2e, D2o, D3, D4, bas, ares,
                eaw, aupd):
    x_ab32 = ab[...]
    x_ab = x_ab32.astype(_BF)
    na_e, na_o = _unpack2(na[...])
    nab_e, nab_o = na_e.astype(_BF), na_o.astype(_BF)
    x_ik = ik[...].astype(_BF)
    x_ij = ij[...].astype(_BF)
    dot = functools.partial(jnp.dot, preferred_element_type=_F32)
    pre_e = (dot(x_ab, C1[...]) + dot(nab_e, C2e[...]) + dot(nab_o, C2o[...])
             + dot(x_ik, C3[...]) + dot(x_ij, C4[...]) + bea[...])
    eaw[...] = _silu(pre_e) * asw[...]
    pre_a = (dot(x_ab, D1[...]) + dot(nab_e, D2e[...]) + dot(nab_o, D2o[...])
             + dot(x_ik, D3[...]) + dot(x_ij, D4[...]) + bas[...])
    aupd[...] = x_ab32 + ares[...] * _silu(pre_a)


def _tc_angle_mlp(angle_ebd, out_n, out_e, a_sw2, Wea1_p, b_ea1,
                  Was_p, b_as, a_res0):
    grid = (NANGLE // EBLK,)
    nblk = _NP // EBLK
    return pl.pallas_call(
        _angle_body,
        grid=grid,
        in_specs=[
            _row_spec(EBLK, (NANGLE, A_DIM)),
            _row_spec(EBLK, (3 * _NP, _PK), off=2 * nblk),
            _row_spec(EBLK, (2 * _NP, E_DIM)),
            _row_spec(EBLK, (2 * _NP, E_DIM), off=nblk),
            _row_spec(EBLK, (NANGLE, 1)),
            _full_spec((A_DIM, E_DIM)),
            _full_spec((_PK, E_DIM)),
            _full_spec((_PK, E_DIM)),
            _full_spec((E_DIM, E_DIM)),
            _full_spec((E_DIM, E_DIM)),
            _full_spec((1, E_DIM)),
            _full_spec((A_DIM, A_DIM)),
            _full_spec((_PK, A_DIM)),
            _full_spec((_PK, A_DIM)),
            _full_spec((E_DIM, A_DIM)),
            _full_spec((E_DIM, A_DIM)),
            _full_spec((1, A_DIM)),
            _full_spec((1, A_DIM)),
        ],
        out_specs=[
            _row_spec(EBLK, (_NP, E_DIM)),
            _row_spec(EBLK, (NANGLE, A_DIM)),
        ],
        out_shape=(
            jax.ShapeDtypeStruct((_NP, E_DIM), _F32),
            jax.ShapeDtypeStruct((NANGLE, A_DIM), _F32),
        ),
    )(angle_ebd, out_n, out_e, out_e, a_sw2, *Wea1_p, b_ea1[None, :],
      *Was_p, b_as[None, :], a_res0[None, :])


def _node_body(xr, her, hnAr, hnBr, mAr, mBr, Wns, bns, Wsym, bsym,
               nr0, nr1, nr2, out):
    x = xr[...]
    dot = functools.partial(jnp.dot, preferred_element_type=_F32)
    node_self = _silu(dot(x.astype(_BF), Wns[...]) + bns[...])
    F = 1.0 / (DYN_E * 3.0)
    he = [her[c] for c in range(3)]
    hnA = [hnAr[c] for c in range(3)]
    hnB = [hnBr[c] for c in range(3)]
    pre = jnp.zeros_like(x) + bsym[...]
    for a in range(AXIS):
        ge_a = (he[0][:, a:a + 1] * he[0] + he[1][:, a:a + 1] * he[1]
                + he[2][:, a:a + 1] * he[2]) * F
        pre += dot(ge_a.astype(_BF), Wsym[a * E_DIM:(a + 1) * E_DIM, :])
        sel = (hnA, hnB)[a % 2]
        j = a // 2
        gnA_a = (sel[0][:, j:j + 1] * hnA[0] + sel[1][:, j:j + 1] * hnA[1]
                 + sel[2][:, j:j + 1] * hnA[2]) * F
        gnB_a = (sel[0][:, j:j + 1] * hnB[0] + sel[1][:, j:j + 1] * hnB[1]
                 + sel[2][:, j:j + 1] * hnB[2]) * F
        base = AXIS * E_DIM + a * N_DIM
        pre += dot(gnA_a.astype(_BF), Wsym[base:base + E_DIM, :])
        pre += dot(gnB_a.astype(_BF), Wsym[base + E_DIM:base + N_DIM, :])
    node_sym = _silu(pre)
    msg = jnp.concatenate([mAr[...], mBr[...]], axis=1)
    out[...] = (x + nr0[...] * node_self + nr1[...] * node_sym
                + nr2[...] * (msg * (1.0 / DYN_E)))


def _tc_node(node_ebd, hge, hgnA, hgnB, msgA, msgB, Wns_bf, b_ns, Wsym_perm,
             b_sym, n_res0, n_res1, n_res2):
    grid = (NLOC // NBLK,)
    return pl.pallas_call(
        _node_body,
        grid=grid,
        in_specs=[
            _row_spec(NBLK, (NLOC, N_DIM)),
            pl.BlockSpec((3, NBLK, E_DIM), lambda i: (0, i, 0)),
            pl.BlockSpec((3, NBLK, E_DIM), lambda i: (0, i, 0)),
            pl.BlockSpec((3, NBLK, E_DIM), lambda i: (0, i, 0)),
            _row_spec(NBLK, (_RN, E_DIM)),
            _row_spec(NBLK, (_RN, E_DIM)),
            _full_spec((N_DIM, N_DIM)),
            _full_spec((1, N_DIM)),
            _full_spec(((N_DIM + E_DIM) * AXIS, N_DIM)),
            _full_spec((1, N_DIM)),
            _full_spec((1, N_DIM)),
            _full_spec((1, N_DIM)),
            _full_spec((1, N_DIM)),
        ],
        out_specs=[_row_spec(NBLK, (NLOC, N_DIM))],
        out_shape=(jax.ShapeDtypeStruct((NLOC, N_DIM), _F32),),
    )(node_ebd, hge, hgnA, hgnB, msgA, msgB, Wns_bf, b_ns[None, :],
      Wsym_perm, b_sym[None, :], n_res0[None, :], n_res1[None, :],
      n_res2[None, :])[0]


def _fin_body(ep, red, W, b, eres, out):
    dot = functools.partial(jnp.dot, preferred_element_type=_F32)
    pre = dot(red[...].astype(_BF), W[...]) * (DYN_A ** -0.5) + b[...]
    out[...] = ep[...] + eres[...] * _silu(pre)


def _tc_edge_fin(e_part, reduced, W_ea2, b_ea2, e_res1):
    grid = (NEDGE // EBLK,)
    return pl.pallas_call(
        _fin_body,
        grid=grid,
        in_specs=[
            _row_spec(EBLK, (_NP, E_DIM)),
            _row_spec(EBLK, (NEDGE, E_DIM)),
            _full_spec((E_DIM, E_DIM)),
            _full_spec((1, E_DIM)),
            _full_spec((1, E_DIM)),
        ],
        out_specs=[_row_spec(EBLK, (NEDGE, E_DIM))],
        out_shape=(jax.ShapeDtypeStruct((NEDGE, E_DIM), _F32),),
    )(e_part, reduced, W_ea2, b_ea2[None, :], e_res1[None, :])[0]


def _deint(W):
    return W[0::2].astype(_BF), W[1::2].astype(_BF)


def kernel(node_ebd_ext, edge_ebd, h2, angle_ebd, nlist, nlist_mask, sw,
           a_nlist, a_nlist_mask, a_sw, edge_index, angle_index, W_ns, b_ns,
           W_sym, b_sym, W_ne, b_ne, W_es, b_es, W_ea1, b_ea1, W_ea2, b_ea2,
           W_as, b_as, n_res0, n_res1, n_res2, e_res0, e_res1, a_res0):
    node_ext_flat = node_ebd_ext.reshape(-1, N_DIM)
    node_ebd = node_ext_flat[:NLOC]
    n2e = edge_index[0]
    nx2e = edge_index[1]
    n2a = angle_index[0]
    eij2a = angle_index[1]
    eik2a = angle_index[2]

    pad = _NP - NEDGE
    node_pack = lax.bitcast_convert_type(
        node_ext_flat.astype(_BF).reshape(NALL, _PK, 2), jnp.int32)

    def _pad0(ix):
        return jnp.pad(ix, (0, pad))

    idx_n = jnp.concatenate([_pad0(nx2e), _pad0(n2e), _pad0(n2a)]
                            ).reshape(_CHN, _C)
    idx_e = jnp.concatenate([_pad0(eik2a), _pad0(eij2a)]).reshape(_CHE, _C)
    idx2d = jnp.pad(n2e, (0, pad), constant_values=_DUMP).reshape(_NCH, _C)
    eij_pad = jnp.pad(eij2a, (0, pad), constant_values=2 * NEDGE)
    ps = jnp.arange(16, dtype=jnp.int32)[:, None]
    loc = eij_pad[None, :].astype(jnp.int32) - ps * _SEG4
    idx16 = jnp.where((loc >= 0) & (loc < _SEG4), loc, _DUMP)
    idx16 = idx16.reshape(16 * _NCH, _C)
    zer = jnp.zeros((_RN, E_DIM), _F32)

    Wne_p = (*_deint(W_ne[:N_DIM]), *_deint(W_ne[N_DIM:2 * N_DIM]),
             W_ne[2 * N_DIM:].astype(_BF))
    Wes_p = (*_deint(W_es[:N_DIM]), *_deint(W_es[N_DIM:2 * N_DIM]),
             W_es[2 * N_DIM:].astype(_BF))
    c0, c1, c2 = A_DIM, A_DIM + N_DIM, A_DIM + N_DIM + E_DIM
    Wea1_p = (W_ea1[:c0].astype(_BF), *_deint(W_ea1[c0:c1]),
              W_ea1[c1:c2].astype(_BF), W_ea1[c2:].astype(_BF))
    Was_p = (W_as[:c0].astype(_BF), *_deint(W_as[c0:c1]),
             W_as[c1:c2].astype(_BF), W_as[c2:].astype(_BF))
    sym_blocks = [W_sym[:AXIS * E_DIM]]
    for a in range(AXIS):
        blk = W_sym[AXIS * E_DIM + a * N_DIM:AXIS * E_DIM + (a + 1) * N_DIM]
        sym_blocks += [blk[0::2], blk[1::2]]
    Wsym_perm = jnp.concatenate(sym_blocks, axis=0).astype(_BF)

    out_n, out_e = _sc_gather(node_pack, edge_ebd, idx_n, idx_e)

    neu, e_part, fh0, fh1, fh2 = _tc_edge_mlp(
        out_n, edge_ebd, sw[:, None], h2, Wne_p, b_ne, Wes_p, b_es, e_res0)
    eaw, a_updated = _tc_angle_mlp(
        angle_ebd, out_n, out_e, a_sw[:, None], Wea1_p, b_ea1, Was_p, b_as,
        a_res0)

    msgA, msgB, hge, hgnA, hgnB = _sc_segsum_node(
        neu, fh0, fh1, fh2, idx2d, zer)
    reduced = _sc_segsum_edge(eaw, idx16, zer)

    n_updated = _tc_node(node_ebd, hge, hgnA, hgnB, msgA, msgB,
                         W_ns.astype(_BF), b_ns, Wsym_perm, b_sym,
                         n_res0, n_res1, n_res2)
    e_updated = _tc_edge_fin(e_part, reduced, W_ea2.astype(_BF), b_ea2,
                             e_res1)

    return (n_updated.reshape(1, NLOC, N_DIM), e_updated, a_updated)
